# baseline (speedup 1.0000x reference)
import jax
import jax.numpy as jnp
from jax.experimental import pallas as pl
from jax.experimental.pallas import tpu as pltpu


def _body(vids_ref, sids_ref, tokens_ref, ve_ref, se_ref, out_ref):
    i = pl.program_id(0)
    vf = vids_ref[i].astype(jnp.float32)
    sf = sids_ref[i].astype(jnp.float32)
    ve0 = ve_ref[0:1, :]
    ve1 = ve_ref[1:2, :]
    se0 = se_ref[0:1, :]
    se1 = se_ref[1:2, :]
    geom = ve0 + vf * (ve1 - ve0) + se0 + sf * (se1 - se0)
    out_ref[:] = tokens_ref[:] + geom


def kernel(tokens, view_ids, side_ids, view_embed, side_embed):
    B, L, D = tokens.shape
    tokens2 = tokens.reshape(B * L, D)
    grid_spec = pltpu.PrefetchScalarGridSpec(
        num_scalar_prefetch=2,
        grid=(B,),
        in_specs=[
            pl.BlockSpec((L, D), lambda i, v, s: (i, 0)),
            pl.BlockSpec((2, D), lambda i, v, s: (0, 0)),
            pl.BlockSpec((2, D), lambda i, v, s: (0, 0)),
        ],
        out_specs=pl.BlockSpec((L, D), lambda i, v, s: (i, 0)),
    )
    out2 = pl.pallas_call(
        _body,
        grid_spec=grid_spec,
        out_shape=jax.ShapeDtypeStruct((B * L, D), tokens.dtype),
    )(view_ids.astype(jnp.int32), side_ids.astype(jnp.int32), tokens2,
      view_embed, side_embed)
    return out2.reshape(B, L, D)


# speedup vs baseline: 8.0802x; 8.0802x over previous
import jax
import jax.numpy as jnp
from jax.experimental import pallas as pl
from jax.experimental.pallas import tpu as pltpu


_B_BLK = 32


def _body(vids_ref, sids_ref, tokens_ref, ve_ref, se_ref, out_ref):
    i = pl.program_id(0)
    L = tokens_ref.shape[0] // _B_BLK
    ve0 = ve_ref[0:1, :]
    ve1 = ve_ref[1:2, :]
    se0 = se_ref[0:1, :]
    se1 = se_ref[1:2, :]

    def row(j, _):
        vf = vids_ref[i * _B_BLK + j].astype(jnp.float32)
        sf = sids_ref[i * _B_BLK + j].astype(jnp.float32)
        geom = ve0 + vf * (ve1 - ve0) + se0 + sf * (se1 - se0)
        sl = pl.ds(j * L, L)
        out_ref[sl, :] = tokens_ref[sl, :] + geom
        return 0

    jax.lax.fori_loop(0, _B_BLK, row, 0)


def kernel(tokens, view_ids, side_ids, view_embed, side_embed):
    B, L, D = tokens.shape
    tokens2 = tokens.reshape(B * L, D)
    rows_blk = _B_BLK * L
    grid_spec = pltpu.PrefetchScalarGridSpec(
        num_scalar_prefetch=2,
        grid=(B // _B_BLK,),
        in_specs=[
            pl.BlockSpec((rows_blk, D), lambda i, v, s: (i, 0)),
            pl.BlockSpec((2, D), lambda i, v, s: (0, 0)),
            pl.BlockSpec((2, D), lambda i, v, s: (0, 0)),
        ],
        out_specs=pl.BlockSpec((rows_blk, D), lambda i, v, s: (i, 0)),
    )
    out2 = pl.pallas_call(
        _body,
        grid_spec=grid_spec,
        out_shape=jax.ShapeDtypeStruct((B * L, D), tokens.dtype),
    )(view_ids.astype(jnp.int32), side_ids.astype(jnp.int32), tokens2,
      view_embed, side_embed)
    return out2.reshape(B, L, D)


# speedup vs baseline: 8.2234x; 1.0177x over previous
import jax
import jax.numpy as jnp
from jax.experimental import pallas as pl
from jax.experimental.pallas import tpu as pltpu


_B_BLK = 32


def _body(vids_ref, sids_ref, tokens_ref, ve_ref, se_ref, out_ref):
    i = pl.program_id(0)
    L = tokens_ref.shape[0] // _B_BLK
    ve0 = ve_ref[0:1, :]
    ve1 = ve_ref[1:2, :]
    se0 = se_ref[0:1, :]
    se1 = se_ref[1:2, :]

    for j in range(_B_BLK):
        vf = vids_ref[i * _B_BLK + j].astype(jnp.float32)
        sf = sids_ref[i * _B_BLK + j].astype(jnp.float32)
        geom = ve0 + vf * (ve1 - ve0) + se0 + sf * (se1 - se0)
        sl = pl.ds(j * L, L)
        out_ref[sl, :] = tokens_ref[sl, :] + geom


def kernel(tokens, view_ids, side_ids, view_embed, side_embed):
    B, L, D = tokens.shape
    tokens2 = tokens.reshape(B * L, D)
    rows_blk = _B_BLK * L
    grid_spec = pltpu.PrefetchScalarGridSpec(
        num_scalar_prefetch=2,
        grid=(B // _B_BLK,),
        in_specs=[
            pl.BlockSpec((rows_blk, D), lambda i, v, s: (i, 0)),
            pl.BlockSpec((2, D), lambda i, v, s: (0, 0)),
            pl.BlockSpec((2, D), lambda i, v, s: (0, 0)),
        ],
        out_specs=pl.BlockSpec((rows_blk, D), lambda i, v, s: (i, 0)),
    )
    out2 = pl.pallas_call(
        _body,
        grid_spec=grid_spec,
        out_shape=jax.ShapeDtypeStruct((B * L, D), tokens.dtype),
    )(view_ids.astype(jnp.int32), side_ids.astype(jnp.int32), tokens2,
      view_embed, side_embed)
    return out2.reshape(B, L, D)
